# baseline (speedup 1.0000x reference)
import jax
import jax.numpy as jnp
from jax.experimental import pallas as pl
from jax.experimental.pallas import tpu as pltpu

_K = 2
_BM_ROUTED = 256
_BM_SHARED = 512
_BM_ROUTER = 512


def _gelu_exact(h):
    return 0.5 * h * (1.0 + jax.lax.erf(h * 0.7071067811865476))


def _router_body(x_ref, w_ref, b_ref, eids_ref, gvals_ref):
    x = x_ref[...]
    w = w_ref[...]
    xh = x.astype(jnp.bfloat16)
    xl = (x - xh.astype(jnp.float32)).astype(jnp.bfloat16)
    wh = w.astype(jnp.bfloat16)
    wl = (w - wh.astype(jnp.float32)).astype(jnp.bfloat16)
    logits = (jnp.dot(xh, wh, preferred_element_type=jnp.float32)
              + jnp.dot(xh, wl, preferred_element_type=jnp.float32)
              + jnp.dot(xl, wh, preferred_element_type=jnp.float32)
              + b_ref[...])
    m = jnp.max(logits, axis=1, keepdims=True)
    ex = jnp.exp(logits - m)
    aff = ex / jnp.sum(ex, axis=1, keepdims=True)
    nr = aff.shape[1]
    iota = jax.lax.broadcasted_iota(jnp.int32, aff.shape, 1)
    m1 = jnp.max(aff, axis=1, keepdims=True)
    i1 = jnp.min(jnp.where(aff == m1, iota, nr), axis=1, keepdims=True)
    aff2 = jnp.where(iota == i1, -1.0, aff)
    m2 = jnp.max(aff2, axis=1, keepdims=True)
    i2 = jnp.min(jnp.where(aff2 == m2, iota, nr), axis=1, keepdims=True)
    eids_ref[...] = jnp.concatenate([i1, i2], axis=1)
    gvals_ref[...] = jnp.concatenate([m1, m2], axis=1)


def _shared_body(xb_ref, w1_ref, b1_ref, w2_ref, b2s_ref, y1_ref, y2_ref,
                 out_ref):
    x = xb_ref[...]
    h = jnp.dot(x, w1_ref[...], preferred_element_type=jnp.float32) + b1_ref[...]
    h = _gelu_exact(h)
    y = jnp.dot(h.astype(jnp.bfloat16), w2_ref[...],
                preferred_element_type=jnp.float32)
    out_ref[...] = (x.astype(jnp.float32) + b2s_ref[...] + y
                    + y1_ref[...].astype(jnp.float32)
                    + y2_ref[...].astype(jnp.float32))


def _grouped_body(eids_ref, x_ref, w1_ref, b1_ref, w2_ref, b2_ref, gate_ref,
                  out_ref):
    del eids_ref
    x = x_ref[...]
    h = jnp.dot(x, w1_ref[0], preferred_element_type=jnp.float32) + b1_ref[0]
    h = _gelu_exact(h)
    y = (jnp.dot(h.astype(jnp.bfloat16), w2_ref[0],
                 preferred_element_type=jnp.float32) + b2_ref[0])
    out_ref[...] = (y * gate_ref[...]).astype(jnp.bfloat16)


def kernel(x, shared_w1, shared_b1, shared_w2, shared_b2,
           routed_w1, routed_b1, routed_w2, routed_b2,
           router_w, router_b):
    B, S, H = x.shape
    NS, _, EI = shared_w1.shape
    NR = router_w.shape[1]
    T = B * S
    P = T * _K

    xf = x.reshape(T, H)
    xb = xf.astype(jnp.bfloat16)
    sw1 = shared_w1.astype(jnp.bfloat16)
    sw2 = shared_w2.astype(jnp.bfloat16)
    rw1 = routed_w1.astype(jnp.bfloat16)
    rw2 = routed_w2.astype(jnp.bfloat16)

    bm_r = min(_BM_ROUTER, T)
    eids, gvals = pl.pallas_call(
        _router_body,
        grid=(T // bm_r,),
        in_specs=[
            pl.BlockSpec((bm_r, H), lambda i: (i, 0)),
            pl.BlockSpec((H, NR), lambda i: (0, 0)),
            pl.BlockSpec((1, NR), lambda i: (0, 0)),
        ],
        out_specs=[
            pl.BlockSpec((bm_r, _K), lambda i: (i, 0)),
            pl.BlockSpec((bm_r, _K), lambda i: (i, 0)),
        ],
        out_shape=[
            jax.ShapeDtypeStruct((T, _K), jnp.int32),
            jax.ShapeDtypeStruct((T, _K), jnp.float32),
        ],
    )(xf, router_w, router_b.reshape(1, NR))

    bm = min(_BM_ROUTED, T)
    e_flat = eids.reshape(P)
    onehot = (e_flat[:, None] == jnp.arange(NR)[None, :]).astype(jnp.int32)
    cum = jnp.cumsum(onehot, axis=0)
    rank = jnp.take_along_axis(cum - onehot, e_flat[:, None], axis=1)[:, 0]
    counts = cum[-1]
    padded = ((counts + bm - 1) // bm) * bm
    offs = jnp.concatenate([jnp.zeros(1, jnp.int32),
                            jnp.cumsum(padded)[:-1].astype(jnp.int32)])
    dst = offs[e_flat] + rank
    NB = P // bm + NR
    Ppad = NB * bm
    pairs = jnp.stack([(jnp.arange(P, dtype=jnp.int32) // _K)
                       .astype(jnp.float32), gvals.reshape(P)], axis=1)
    scat = jnp.zeros((Ppad, 2), jnp.float32).at[dst].set(pairs)
    token_src = scat[:, 0].astype(jnp.int32)
    gate_sorted = scat[:, 1:2]
    block_eids = jnp.repeat(jnp.arange(NR, dtype=jnp.int32), padded // bm,
                            total_repeat_length=NB)
    x_sorted = jax.lax.optimization_barrier(xb)[token_src]

    y_sorted = pl.pallas_call(
        _grouped_body,
        grid_spec=pltpu.PrefetchScalarGridSpec(
            num_scalar_prefetch=1,
            grid=(NB,),
            in_specs=[
                pl.BlockSpec((bm, H), lambda i, eids: (i, 0)),
                pl.BlockSpec((1, H, EI), lambda i, eids: (eids[i], 0, 0)),
                pl.BlockSpec((1, 1, EI), lambda i, eids: (eids[i], 0, 0)),
                pl.BlockSpec((1, EI, H), lambda i, eids: (eids[i], 0, 0)),
                pl.BlockSpec((1, 1, H), lambda i, eids: (eids[i], 0, 0)),
                pl.BlockSpec((bm, 1), lambda i, eids: (i, 0)),
            ],
            out_specs=pl.BlockSpec((bm, H), lambda i, eids: (i, 0)),
        ),
        out_shape=jax.ShapeDtypeStruct((Ppad, H), jnp.bfloat16),
    )(block_eids, x_sorted, rw1, routed_b1.reshape(NR, 1, EI), rw2,
      routed_b2.reshape(NR, 1, H), gate_sorted)

    pos = dst.reshape(T, _K)
    y1 = jax.lax.optimization_barrier(y_sorted[pos[:, 0]])
    y2 = jax.lax.optimization_barrier(y_sorted[pos[:, 1]])

    bm_s = min(_BM_SHARED, T)
    w1cat = sw1.transpose(1, 0, 2).reshape(H, NS * EI)
    w2cat = sw2.reshape(NS * EI, H)
    b1cat = shared_b1.reshape(1, NS * EI)
    b2s = jnp.sum(shared_b2, axis=0).reshape(1, H)
    out = pl.pallas_call(
        _shared_body,
        grid=(T // bm_s,),
        in_specs=[
            pl.BlockSpec((bm_s, H), lambda i: (i, 0)),
            pl.BlockSpec((H, NS * EI), lambda i: (0, 0),
                         pipeline_mode=pl.Buffered(buffer_count=1)),
            pl.BlockSpec((1, NS * EI), lambda i: (0, 0)),
            pl.BlockSpec((NS * EI, H), lambda i: (0, 0),
                         pipeline_mode=pl.Buffered(buffer_count=1)),
            pl.BlockSpec((1, H), lambda i: (0, 0)),
            pl.BlockSpec((bm_s, H), lambda i: (i, 0)),
            pl.BlockSpec((bm_s, H), lambda i: (i, 0)),
        ],
        out_specs=pl.BlockSpec((bm_s, H), lambda i: (i, 0)),
        out_shape=jax.ShapeDtypeStruct((T, H), jnp.float32),
    )(xb, w1cat, b1cat, w2cat, b2s, y1, y2)
    return out.reshape(B, S, H)
